# baseline (speedup 1.0000x reference)
import functools

import jax
import jax.numpy as jnp
from jax import lax
from jax.experimental import pallas as pl
from jax.experimental.pallas import tpu as pltpu
from jax.experimental.pallas import tpu_sc as plsc

N = 10000
D = 128
E = 320000
L = 4
NUM_GRAPHS = 64
NUM_CLASSES = 10

NC = 2
NS = 16
LANE = 16
NW = NC * NS
N_PAD = 10240
CH = 64
EPW = 10240
NCH = EPW // CH
E_PAD = NW * EPW
RPS = N_PAD // NS


NBUF = 4


def _sc_edge_body(h_hbm, a_s_hbm, a_d_hbm, packed_hbm,
                  u_out, g_out,
                  packed_v, srcix_v, dstix_v, asrc_v, adst_v, gate_v, rows_v,
                  zrow_v, gz_v, u_sh, g_sh, sem_g, sem_g2, sem_su, sem_sg):
    c = lax.axis_index("c")
    s = lax.axis_index("s")
    wid = c * NS + s

    zero16 = jnp.zeros((LANE,), jnp.float32)
    mask14 = jnp.full((LANE,), 0x3FFF, jnp.int32)
    sh14 = jnp.full((LANE,), 14, jnp.int32)

    def zrow_body(r, carry):
        for q in range(D // LANE):
            zrow_v[r, pl.ds(q * LANE, LANE)] = zero16
        return carry
    lax.fori_loop(0, 8, zrow_body, 0)

    def gz_body(i, carry):
        gz_v[pl.ds(i * LANE, LANE)] = zero16
        return carry
    lax.fori_loop(0, RPS // LANE, gz_body, 0)

    def uz_body(k, carry):
        pltpu.sync_copy(zrow_v, u_sh.at[pl.ds(s * RPS + k * 8, 8)])
        return carry
    lax.fori_loop(0, RPS // 8, uz_body, 0)
    pltpu.sync_copy(gz_v, g_sh.at[pl.ds(s * RPS, RPS)])
    plsc.subcore_barrier()

    pltpu.sync_copy(packed_hbm.at[wid], packed_v)

    def issue_gathers(j, b):
        jrow = j // 2
        jcol = (j % 2) * CH
        for q in range(CH // LANE):
            sl = pl.ds(q * LANE, LANE)
            p = packed_v[jrow, pl.ds(jcol + q * LANE, LANE)]
            srcix_v[b, sl] = lax.bitwise_and(p, mask14)
            dstix_v[b, sl] = lax.shift_right_logical(p, sh14)
        idx_s = srcix_v.at[b]
        idx_d = dstix_v.at[b]
        pltpu.async_copy(h_hbm.at[idx_s.at[pl.ds(0, CH // 2)]],
                         rows_v.at[b, pl.ds(0, CH // 2)], sem_g.at[b])
        pltpu.async_copy(h_hbm.at[idx_s.at[pl.ds(CH // 2, CH // 2)]],
                         rows_v.at[b, pl.ds(CH // 2, CH // 2)], sem_g2.at[b])
        pltpu.async_copy(a_s_hbm.at[idx_s], asrc_v.at[b], sem_g.at[b])
        pltpu.async_copy(a_d_hbm.at[idx_d], adst_v.at[b], sem_g.at[b])

    def wait_gathers(b):
        idx_s = srcix_v.at[b]
        idx_d = dstix_v.at[b]
        pltpu.make_async_copy(h_hbm.at[idx_s.at[pl.ds(0, CH // 2)]],
                              rows_v.at[b, pl.ds(0, CH // 2)], sem_g.at[b]).wait()
        pltpu.make_async_copy(h_hbm.at[idx_s.at[pl.ds(CH // 2, CH // 2)]],
                              rows_v.at[b, pl.ds(CH // 2, CH // 2)], sem_g2.at[b]).wait()
        pltpu.make_async_copy(a_s_hbm.at[idx_s], asrc_v.at[b], sem_g.at[b]).wait()
        pltpu.make_async_copy(a_d_hbm.at[idx_d], adst_v.at[b], sem_g.at[b]).wait()

    def wait_scatters(b):
        idx_d = dstix_v.at[b]
        pltpu.make_async_copy(rows_v.at[b], u_sh.at[idx_d], sem_su.at[b]).wait()
        pltpu.make_async_copy(gate_v.at[b], g_sh.at[idx_d], sem_sg.at[b]).wait()

    issue_gathers(0, 0)

    def group_body(g, carry):
        for b in range(NBUF):
            j = g * NBUF + b
            bn = (b + 1) % NBUF

            @pl.when(j >= NBUF - 1)
            def _():
                wait_scatters(bn)

            @pl.when(j + 1 < NCH)
            def _():
                issue_gathers(j + 1, bn)

            wait_gathers(b)
            for q in range(CH // LANE):
                sl = pl.ds(q * LANE, LANE)
                t = asrc_v[b, sl] + adst_v[b, sl]
                gate_v[b, sl] = 1.0 / (1.0 + jnp.exp(-t))


            idx_d = dstix_v.at[b]
            pltpu.async_copy(rows_v.at[b], u_sh.at[idx_d], sem_su.at[b], add=True)
            pltpu.async_copy(gate_v.at[b], g_sh.at[idx_d], sem_sg.at[b], add=True)
        return carry
    lax.fori_loop(0, NCH // NBUF, group_body, 0)

    for jj in range(NCH - (NBUF - 1), NCH):
        wait_scatters(jj % NBUF)

    plsc.subcore_barrier()
    base = s * RPS
    pltpu.sync_copy(u_sh.at[pl.ds(base, RPS)], u_out.at[c, pl.ds(base, RPS)])
    pltpu.sync_copy(g_sh.at[pl.ds(base, RPS)], g_out.at[c, pl.ds(base, RPS)])


@functools.cache
def _make_sc_edges():
  return pl.kernel(
    _sc_edge_body,
    out_type=[jax.ShapeDtypeStruct((NC, N_PAD, D), jnp.float32),
              jax.ShapeDtypeStruct((NC, N_PAD), jnp.float32)],
    mesh=plsc.VectorSubcoreMesh(core_axis_name="c", subcore_axis_name="s",
                                num_cores=NC, num_subcores=NS),
    scratch_types=[
        pltpu.VMEM((NCH // 2, 2 * CH), jnp.int32),
        pltpu.VMEM((NBUF, CH), jnp.int32),
        pltpu.VMEM((NBUF, CH), jnp.int32),
        pltpu.VMEM((NBUF, CH), jnp.float32),
        pltpu.VMEM((NBUF, CH), jnp.float32),
        pltpu.VMEM((NBUF, CH), jnp.float32),
        pltpu.VMEM((NBUF, CH, D), jnp.float32),
        pltpu.VMEM((8, D), jnp.float32),
        pltpu.VMEM((RPS,), jnp.float32),
        pltpu.VMEM_SHARED((N_PAD, D), jnp.float32),
        pltpu.VMEM_SHARED((N_PAD,), jnp.float32),
        pltpu.SemaphoreType.DMA((NBUF,)),
        pltpu.SemaphoreType.DMA((NBUF,)),
        pltpu.SemaphoreType.DMA((NBUF,)),
        pltpu.SemaphoreType.DMA((NBUF,)),
    ],
  )


def _tc_pre_body(x_ref, win_ref, bin_ref, wab_ref, bab_ref, h_ref, a_ref):
    h = jnp.maximum(x_ref[...] @ win_ref[...] + bin_ref[...], 0.0)
    h_ref[...] = h
    a_ref[...] = h @ wab_ref[...] + bab_ref[...]


_tc_pre = pl.pallas_call(
    _tc_pre_body,
    out_shape=[jax.ShapeDtypeStruct((N_PAD, D), jnp.float32),
               jax.ShapeDtypeStruct((N_PAD, 2), jnp.float32)],
)


def _tc_layer_body(u_ref, g_ref, h_ref, wm_ref, bm_ref, wr_ref, br_ref,
                   gam_ref, bet_ref, wab_ref, bab_ref, hn_ref, a_ref):
    u = u_ref[0] + u_ref[1]
    gcol = g_ref[0] + g_ref[1]
    h = h_ref[...]
    z = (u @ wm_ref[...] + gcol * bm_ref[...]
         + h @ wr_ref[...] + br_ref[...])
    rows = lax.broadcasted_iota(jnp.int32, (N_PAD, 1), 0)
    mask = rows < N
    zm = jnp.where(mask, z, 0.0)
    mean = jnp.sum(zm, axis=0, keepdims=True) * (1.0 / N)
    dev = jnp.where(mask, z - mean, 0.0)
    var = jnp.sum(dev * dev, axis=0, keepdims=True) * (1.0 / N)
    zn = (z - mean) * lax.rsqrt(var + 1e-5) * gam_ref[...] + bet_ref[...]
    hn = jnp.maximum(zn, 0.0)
    hn_ref[...] = hn
    a_ref[...] = hn @ wab_ref[...] + bab_ref[...]


_tc_layer = pl.pallas_call(
    _tc_layer_body,
    out_shape=[jax.ShapeDtypeStruct((N_PAD, D), jnp.float32),
               jax.ShapeDtypeStruct((N_PAD, 2), jnp.float32)],
)


def _tc_final_body(h_ref, batch_ref, w1_ref, b1_ref, w2_ref, b2_ref, out_ref):
    gid = lax.broadcasted_iota(jnp.int32, (NUM_GRAPHS, N_PAD), 0)
    onehot_t = (gid == batch_ref[...]).astype(jnp.float32)
    pooled = onehot_t @ h_ref[...]
    hid = jnp.maximum(pooled @ w1_ref[...] + b1_ref[...], 0.0)
    out_ref[...] = (hid @ w2_ref[...] + b2_ref[...]) * 0.5


_tc_final = pl.pallas_call(
    _tc_final_body,
    out_shape=jax.ShapeDtypeStruct((NUM_GRAPHS, NUM_CLASSES), jnp.float32),
)


def kernel(x, edge_index, batch, W_in, b_in, Wa, ba, Wm, bm, Wr, br,
           gamma, beta, W1, b1, W2, b2):
    f32 = jnp.float32
    src = edge_index[0].astype(jnp.int32)
    dst = edge_index[1].astype(jnp.int32)
    pad_e = E_PAD - E
    packed = src | (dst << 14)
    packed_p = jnp.concatenate(
        [packed, jnp.full((pad_e,), N << 14, jnp.int32)]).reshape(NW, NCH // 2, 2 * CH)
    x_p = jnp.concatenate([x.astype(f32), jnp.zeros((N_PAD - N, D), f32)], axis=0)
    batch2 = jnp.concatenate(
        [batch.astype(jnp.int32),
         jnp.full((N_PAD - N,), NUM_GRAPHS, jnp.int32)]).reshape(1, N_PAD)

    wabs, babs = [], []
    for i in range(L):
        wabs.append(jnp.concatenate([Wa[i, :D, :], Wa[i, D:, :]], axis=1))
        babs.append(jnp.concatenate(
            [jnp.zeros((1,), f32), ba[i]]).reshape(1, 2))
    wabs.append(jnp.zeros((D, 2), f32))
    babs.append(jnp.zeros((1, 2), f32))

    h, A = _tc_pre(x_p, W_in, b_in.reshape(1, D), wabs[0], babs[0])
    for i in range(L):
        a_s = A[:, 0]
        a_d = A[:, 1]
        u_part, g_part = _make_sc_edges()(h, a_s, a_d, packed_p)
        h, A = _tc_layer(u_part, g_part.reshape(NC, N_PAD, 1), h,
                         Wm[i], bm[i].reshape(1, D),
                         Wr[i], br[i].reshape(1, D),
                         gamma[i].reshape(1, D), beta[i].reshape(1, D),
                         wabs[i + 1], babs[i + 1])
    logits = _tc_final(h, batch2, W1, b1.reshape(1, D // 2),
                       W2, b2.reshape(1, NUM_CLASSES))
    return logits


# speedup vs baseline: 1.3850x; 1.3850x over previous
import functools

import jax
import jax.numpy as jnp
import numpy as np
from jax import lax
from jax.experimental import pallas as pl
from jax.experimental.pallas import tpu as pltpu
from jax.experimental.pallas import tpu_sc as plsc

N = 10000
D = 128
E = 320000
L = 4
NUM_GRAPHS = 64
NUM_CLASSES = 10

NC = 2
NS = 16
LANE = 16
NW = NC * NS
N_PAD = 10240
CH = 64
EPW = 10240
NCH = EPW // CH
E_PAD = NW * EPW
RPS = N_PAD // NS


NBUF = 4


def _sc_edge_body(hbi_hbm, a_s_hbm, a_d_hbm, packed_hbm,
                  u_out, g_out,
                  packed_v, srcix_v, dstix_v, asrc_v, adst_v, gate_v,
                  rows_bf, rows_f, zrow_v, gz_v, u_sh, g_sh,
                  sem_g, sem_su, sem_sg):
    c = lax.axis_index("c")
    s = lax.axis_index("s")
    wid = c * NS + s

    zero16 = jnp.zeros((LANE,), jnp.float32)
    mask14 = jnp.full((LANE,), 0x3FFF, jnp.int32)
    sh14 = jnp.full((LANE,), 14, jnp.int32)
    sh16 = jnp.full((LANE,), 16, jnp.int32)
    maskhi = jnp.full((LANE,), -65536, jnp.int32)

    def zrow_body(r, carry):
        for q in range(D // LANE):
            zrow_v[r, pl.ds(q * LANE, LANE)] = zero16
        return carry
    lax.fori_loop(0, 8, zrow_body, 0)

    def gz_body(i, carry):
        gz_v[pl.ds(i * LANE, LANE)] = zero16
        return carry
    lax.fori_loop(0, RPS // LANE, gz_body, 0)

    def uz_body(k, carry):
        pltpu.sync_copy(zrow_v, u_sh.at[pl.ds(s * RPS + k * 8, 8)])
        return carry
    lax.fori_loop(0, RPS // 8, uz_body, 0)
    pltpu.sync_copy(gz_v, g_sh.at[pl.ds(s * RPS, RPS)])
    plsc.subcore_barrier()

    pltpu.sync_copy(packed_hbm.at[wid], packed_v)

    def issue_gathers(j, b):
        jrow = j // 2
        jcol = (j % 2) * CH
        for q in range(CH // LANE):
            sl = pl.ds(q * LANE, LANE)
            p = packed_v[jrow, pl.ds(jcol + q * LANE, LANE)]
            srcix_v[b, sl] = lax.bitwise_and(p, mask14)
            dstix_v[b, sl] = lax.shift_right_logical(p, sh14)
        idx_s = srcix_v.at[b]
        idx_d = dstix_v.at[b]
        pltpu.async_copy(hbi_hbm.at[idx_s], rows_bf.at[b], sem_g.at[b])
        pltpu.async_copy(a_s_hbm.at[idx_s], asrc_v.at[b], sem_g.at[b])
        pltpu.async_copy(a_d_hbm.at[idx_d], adst_v.at[b], sem_g.at[b])

    def wait_gathers(b):
        idx_s = srcix_v.at[b]
        idx_d = dstix_v.at[b]
        pltpu.make_async_copy(hbi_hbm.at[idx_s], rows_bf.at[b], sem_g.at[b]).wait()
        pltpu.make_async_copy(a_s_hbm.at[idx_s], asrc_v.at[b], sem_g.at[b]).wait()
        pltpu.make_async_copy(a_d_hbm.at[idx_d], adst_v.at[b], sem_g.at[b]).wait()

    def wait_row_scatter(bs):
        pltpu.make_async_copy(rows_f.at[bs], u_sh.at[dstix_v.at[0]],
                              sem_su.at[bs]).wait()

    def wait_gate_scatter(b):
        pltpu.make_async_copy(gate_v.at[b], g_sh.at[dstix_v.at[b]],
                              sem_sg.at[b]).wait()

    issue_gathers(0, 0)

    def group_body(g, carry):
        for b in range(NBUF):
            j = g * NBUF + b
            bn = (b + 1) % NBUF
            bs = j % 2

            @pl.when(j >= NBUF - 1)
            def _():
                wait_gate_scatter(bn)

            @pl.when(j + 1 < NCH)
            def _():
                issue_gathers(j + 1, bn)

            @pl.when(j >= 2)
            def _():
                wait_row_scatter(bs)

            wait_gathers(b)
            for q in range(CH // LANE):
                sl = pl.ds(q * LANE, LANE)
                t = asrc_v[b, sl] + adst_v[b, sl]
                gate_v[b, sl] = 1.0 / (1.0 + jnp.exp(-t))

            for q in range(CH // LANE):
                g16 = gate_v[b, pl.ds(q * LANE, LANE)]
                for l in range(LANE):
                    gb = lax.broadcast(g16[l], (LANE,))
                    row = q * LANE + l
                    for cc in range(D // 32):
                        w = rows_bf[b, row, pl.ds(cc * LANE, LANE)]
                        ev = lax.bitcast_convert_type(
                            lax.shift_left(w, sh16), jnp.float32)
                        od = lax.bitcast_convert_type(
                            lax.bitwise_and(w, maskhi), jnp.float32)
                        rows_f[bs, row, pl.ds(cc * 32, LANE)] = ev * gb
                        rows_f[bs, row, pl.ds(cc * 32 + LANE, LANE)] = od * gb

            idx_d = dstix_v.at[b]
            pltpu.async_copy(rows_f.at[bs], u_sh.at[idx_d], sem_su.at[bs], add=True)
            pltpu.async_copy(gate_v.at[b], g_sh.at[idx_d], sem_sg.at[b], add=True)
        return carry
    lax.fori_loop(0, NCH // NBUF, group_body, 0)

    wait_row_scatter((NCH - 2) % 2)
    wait_row_scatter((NCH - 1) % 2)
    for jj in range(NCH - (NBUF - 1), NCH):
        wait_gate_scatter(jj % NBUF)

    plsc.subcore_barrier()
    base = s * RPS
    pltpu.sync_copy(u_sh.at[pl.ds(base, RPS)], u_out.at[c, pl.ds(base, RPS)])
    pltpu.sync_copy(g_sh.at[pl.ds(base, RPS)], g_out.at[c, pl.ds(base, RPS)])


@functools.cache
def _make_sc_edges():
  return pl.kernel(
    _sc_edge_body,
    out_type=[jax.ShapeDtypeStruct((NC, N_PAD, D), jnp.float32),
              jax.ShapeDtypeStruct((NC, N_PAD), jnp.float32)],
    mesh=plsc.VectorSubcoreMesh(core_axis_name="c", subcore_axis_name="s",
                                num_cores=NC, num_subcores=NS),
    compiler_params=pltpu.CompilerParams(use_tc_tiling_on_sc=False),
    scratch_types=[
        pltpu.VMEM((NCH // 2, 2 * CH), jnp.int32),
        pltpu.VMEM((NBUF, CH), jnp.int32),
        pltpu.VMEM((NBUF, CH), jnp.int32),
        pltpu.VMEM((NBUF, CH), jnp.float32),
        pltpu.VMEM((NBUF, CH), jnp.float32),
        pltpu.VMEM((NBUF, CH), jnp.float32),
        pltpu.VMEM((NBUF, CH, D // 2), jnp.int32),
        pltpu.VMEM((2, CH, D), jnp.float32),
        pltpu.VMEM((8, D), jnp.float32),
        pltpu.VMEM((RPS,), jnp.float32),
        pltpu.VMEM_SHARED((N_PAD, D), jnp.float32),
        pltpu.VMEM_SHARED((N_PAD,), jnp.float32),
        pltpu.SemaphoreType.DMA((NBUF,)),
        pltpu.SemaphoreType.DMA((2,)),
        pltpu.SemaphoreType.DMA((NBUF,)),
    ],
  )


def _tc_pre_body(x_ref, win_ref, bin_ref, wab_ref, bab_ref,
                 h_ref, a_ref, hb_ref):
    h = jnp.maximum(x_ref[...] @ win_ref[...] + bin_ref[...], 0.0)
    h_ref[...] = h
    a_ref[...] = h @ wab_ref[...] + bab_ref[...]
    hb_ref[...] = h.astype(jnp.bfloat16)


_tc_pre = pl.pallas_call(
    _tc_pre_body,
    out_shape=[jax.ShapeDtypeStruct((N_PAD, D), jnp.float32),
               jax.ShapeDtypeStruct((N_PAD, 2), jnp.float32),
               jax.ShapeDtypeStruct((N_PAD, D), jnp.bfloat16)],
)


def _tc_layer_body(u_ref, g_ref, h_ref, wm_ref, bm_ref, wr_ref, br_ref,
                   gam_ref, bet_ref, wab_ref, bab_ref, hn_ref, a_ref, hb_ref):
    u = u_ref[0] + u_ref[1]
    gcol = g_ref[0] + g_ref[1]
    h = h_ref[...]
    z = (u @ wm_ref[...] + gcol * bm_ref[...]
         + h @ wr_ref[...] + br_ref[...])
    rows = lax.broadcasted_iota(jnp.int32, (N_PAD, 1), 0)
    mask = rows < N
    zm = jnp.where(mask, z, 0.0)
    mean = jnp.sum(zm, axis=0, keepdims=True) * (1.0 / N)
    dev = jnp.where(mask, z - mean, 0.0)
    var = jnp.sum(dev * dev, axis=0, keepdims=True) * (1.0 / N)
    zn = (z - mean) * lax.rsqrt(var + 1e-5) * gam_ref[...] + bet_ref[...]
    hn = jnp.maximum(zn, 0.0)
    hn_ref[...] = hn
    a_ref[...] = hn @ wab_ref[...] + bab_ref[...]
    hb_ref[...] = hn.astype(jnp.bfloat16)


_tc_layer = pl.pallas_call(
    _tc_layer_body,
    out_shape=[jax.ShapeDtypeStruct((N_PAD, D), jnp.float32),
               jax.ShapeDtypeStruct((N_PAD, 2), jnp.float32),
               jax.ShapeDtypeStruct((N_PAD, D), jnp.bfloat16)],
)


def _tc_final_body(h_ref, batch_ref, w1_ref, b1_ref, w2_ref, b2_ref, out_ref):
    gid = lax.broadcasted_iota(jnp.int32, (NUM_GRAPHS, N_PAD), 0)
    onehot_t = (gid == batch_ref[...]).astype(jnp.float32)
    pooled = onehot_t @ h_ref[...]
    hid = jnp.maximum(pooled @ w1_ref[...] + b1_ref[...], 0.0)
    out_ref[...] = (hid @ w2_ref[...] + b2_ref[...]) * 0.5


_tc_final = pl.pallas_call(
    _tc_final_body,
    out_shape=jax.ShapeDtypeStruct((NUM_GRAPHS, NUM_CLASSES), jnp.float32),
)


def kernel(x, edge_index, batch, W_in, b_in, Wa, ba, Wm, bm, Wr, br,
           gamma, beta, W1, b1, W2, b2):
    f32 = jnp.float32
    src = edge_index[0].astype(jnp.int32)
    dst = edge_index[1].astype(jnp.int32)
    pad_e = E_PAD - E
    packed = src | (dst << 14)
    packed_p = jnp.concatenate(
        [packed, jnp.full((pad_e,), N << 14, jnp.int32)]).reshape(NW, NCH // 2, 2 * CH)
    x_p = jnp.concatenate([x.astype(f32), jnp.zeros((N_PAD - N, D), f32)], axis=0)
    batch2 = jnp.concatenate(
        [batch.astype(jnp.int32),
         jnp.full((N_PAD - N,), NUM_GRAPHS, jnp.int32)]).reshape(1, N_PAD)

    wabs, babs = [], []
    for i in range(L):
        wabs.append(jnp.concatenate([Wa[i, :D, :], Wa[i, D:, :]], axis=1))
        babs.append(jnp.concatenate(
            [jnp.zeros((1,), f32), ba[i]]).reshape(1, 2))
    wabs.append(jnp.zeros((D, 2), f32))
    babs.append(jnp.zeros((1, 2), f32))

    sigma = np.concatenate([
        np.concatenate([np.arange(32 * q, 32 * q + 32, 2),
                        np.arange(32 * q + 1, 32 * q + 32, 2)])
        for q in range(D // 32)])
    h, A, hb = _tc_pre(x_p, W_in, b_in.reshape(1, D), wabs[0], babs[0])
    for i in range(L):
        a_s = A[:, 0]
        a_d = A[:, 1]
        hbi = lax.bitcast_convert_type(
            hb.reshape(N_PAD, D // 2, 2), jnp.int32)
        u_part, g_part = _make_sc_edges()(hbi, a_s, a_d, packed_p)
        h, A, hb = _tc_layer(u_part, g_part.reshape(NC, N_PAD, 1), h,
                         Wm[i][sigma, :], bm[i].reshape(1, D),
                         Wr[i], br[i].reshape(1, D),
                         gamma[i].reshape(1, D), beta[i].reshape(1, D),
                         wabs[i + 1], babs[i + 1])
    logits = _tc_final(h, batch2, W1, b1.reshape(1, D // 2),
                       W2, b2.reshape(1, NUM_CLASSES))
    return logits


# speedup vs baseline: 1.4065x; 1.0155x over previous
import functools

import jax
import jax.numpy as jnp
import numpy as np
from jax import lax
from jax.experimental import pallas as pl
from jax.experimental.pallas import tpu as pltpu
from jax.experimental.pallas import tpu_sc as plsc

N = 10000
D = 128
E = 320000
L = 4
NUM_GRAPHS = 64
NUM_CLASSES = 10

NC = 2
NS = 16
LANE = 16
NW = NC * NS
N_PAD = 10240
CH = 64
EPW = 10240
NCH = EPW // CH
E_PAD = NW * EPW
RPS = N_PAD // NS


NBUF = 4


def _sc_edge_body(hbi_hbm, a_s_hbm, a_d_hbm, packed_hbm,
                  u_out, g_out,
                  packed_v, srcix_v, dstix_v, asrc_v, adst_v, gate_v,
                  rows_bf, rows_f, zrow_v, gz_v, u_sh, g_sh,
                  sem_g, sem_su, sem_sg):
    c = lax.axis_index("c")
    s = lax.axis_index("s")
    wid = c * NS + s

    zero16 = jnp.zeros((LANE,), jnp.float32)
    mask14 = jnp.full((LANE,), 0x3FFF, jnp.int32)
    sh14 = jnp.full((LANE,), 14, jnp.int32)
    sh16 = jnp.full((LANE,), 16, jnp.int32)
    maskhi = jnp.full((LANE,), -65536, jnp.int32)

    def zrow_body(r, carry):
        for q in range(D // LANE):
            zrow_v[r, pl.ds(q * LANE, LANE)] = zero16
        return carry
    lax.fori_loop(0, 8, zrow_body, 0)

    def gz_body(i, carry):
        gz_v[pl.ds(i * LANE, LANE)] = zero16
        return carry
    lax.fori_loop(0, RPS // LANE, gz_body, 0)

    def uz_body(k, carry):
        pltpu.sync_copy(zrow_v, u_sh.at[pl.ds(s * RPS + k * 8, 8)])
        return carry
    lax.fori_loop(0, RPS // 8, uz_body, 0)
    pltpu.sync_copy(gz_v, g_sh.at[pl.ds(s * RPS, RPS)])
    plsc.subcore_barrier()

    pltpu.sync_copy(packed_hbm.at[wid], packed_v)

    def issue_gathers(j, b):
        jrow = j // 2
        jcol = (j % 2) * CH
        for q in range(CH // LANE):
            sl = pl.ds(q * LANE, LANE)
            p = packed_v[jrow, pl.ds(jcol + q * LANE, LANE)]
            srcix_v[b, sl] = lax.bitwise_and(p, mask14)
            dstix_v[b, sl] = lax.shift_right_logical(p, sh14)
        idx_s = srcix_v.at[b]
        idx_d = dstix_v.at[b]
        pltpu.async_copy(hbi_hbm.at[idx_s], rows_bf.at[b], sem_g.at[b])
        pltpu.async_copy(a_s_hbm.at[idx_s], asrc_v.at[b], sem_g.at[b])
        pltpu.async_copy(a_d_hbm.at[idx_d], adst_v.at[b], sem_g.at[b])

    def wait_gathers(b):
        idx_s = srcix_v.at[b]
        idx_d = dstix_v.at[b]
        pltpu.make_async_copy(hbi_hbm.at[idx_s], rows_bf.at[b], sem_g.at[b]).wait()
        pltpu.make_async_copy(a_s_hbm.at[idx_s], asrc_v.at[b], sem_g.at[b]).wait()
        pltpu.make_async_copy(a_d_hbm.at[idx_d], adst_v.at[b], sem_g.at[b]).wait()

    def wait_row_scatter(bs):
        pltpu.make_async_copy(rows_f.at[bs], u_sh.at[dstix_v.at[0]],
                              sem_su.at[bs]).wait()

    def wait_gate_scatter(b):
        pltpu.make_async_copy(gate_v.at[b], g_sh.at[dstix_v.at[b]],
                              sem_sg.at[b]).wait()

    issue_gathers(0, 0)
    issue_gathers(1, 1)

    def group_body(g, carry):
        for b in range(NBUF):
            j = g * NBUF + b
            bn2 = (b + 2) % NBUF
            bs = j % 2

            @pl.when(j >= 2)
            def _():
                wait_gate_scatter(bn2)
                wait_row_scatter(bs)

            @pl.when(j + 2 < NCH)
            def _():
                issue_gathers(j + 2, bn2)

            wait_gathers(b)
            for q in range(CH // LANE):
                sl = pl.ds(q * LANE, LANE)
                t = asrc_v[b, sl] + adst_v[b, sl]
                gate_v[b, sl] = 1.0 / (1.0 + jnp.exp(-t))

            for q in range(CH // LANE):
                g16 = gate_v[b, pl.ds(q * LANE, LANE)]
                for l in range(LANE):
                    gb = lax.broadcast(g16[l], (LANE,))
                    row = q * LANE + l
                    for cc in range(D // 32):
                        w = rows_bf[b, row, pl.ds(cc * LANE, LANE)]
                        ev = lax.bitcast_convert_type(
                            lax.shift_left(w, sh16), jnp.float32)
                        od = lax.bitcast_convert_type(
                            lax.bitwise_and(w, maskhi), jnp.float32)
                        rows_f[bs, row, pl.ds(cc * 32, LANE)] = ev * gb
                        rows_f[bs, row, pl.ds(cc * 32 + LANE, LANE)] = od * gb

            idx_d = dstix_v.at[b]
            pltpu.async_copy(rows_f.at[bs], u_sh.at[idx_d], sem_su.at[bs], add=True)
            pltpu.async_copy(gate_v.at[b], g_sh.at[idx_d], sem_sg.at[b], add=True)
        return carry
    lax.fori_loop(0, NCH // NBUF, group_body, 0)

    wait_row_scatter((NCH - 2) % 2)
    wait_row_scatter((NCH - 1) % 2)
    wait_gate_scatter((NCH - 2) % NBUF)
    wait_gate_scatter((NCH - 1) % NBUF)

    plsc.subcore_barrier()
    base = s * RPS
    pltpu.sync_copy(u_sh.at[pl.ds(base, RPS)], u_out.at[c, pl.ds(base, RPS)])
    pltpu.sync_copy(g_sh.at[pl.ds(base, RPS)], g_out.at[c, pl.ds(base, RPS)])


@functools.cache
def _make_sc_edges():
  return pl.kernel(
    _sc_edge_body,
    out_type=[jax.ShapeDtypeStruct((NC, N_PAD, D), jnp.float32),
              jax.ShapeDtypeStruct((NC, N_PAD), jnp.float32)],
    mesh=plsc.VectorSubcoreMesh(core_axis_name="c", subcore_axis_name="s",
                                num_cores=NC, num_subcores=NS),
    compiler_params=pltpu.CompilerParams(use_tc_tiling_on_sc=False),
    scratch_types=[
        pltpu.VMEM((NCH // 2, 2 * CH), jnp.int32),
        pltpu.VMEM((NBUF, CH), jnp.int32),
        pltpu.VMEM((NBUF, CH), jnp.int32),
        pltpu.VMEM((NBUF, CH), jnp.float32),
        pltpu.VMEM((NBUF, CH), jnp.float32),
        pltpu.VMEM((NBUF, CH), jnp.float32),
        pltpu.VMEM((NBUF, CH, D // 2), jnp.int32),
        pltpu.VMEM((2, CH, D), jnp.float32),
        pltpu.VMEM((8, D), jnp.float32),
        pltpu.VMEM((RPS,), jnp.float32),
        pltpu.VMEM_SHARED((N_PAD, D), jnp.float32),
        pltpu.VMEM_SHARED((N_PAD,), jnp.float32),
        pltpu.SemaphoreType.DMA((NBUF,)),
        pltpu.SemaphoreType.DMA((2,)),
        pltpu.SemaphoreType.DMA((NBUF,)),
    ],
  )


def _tc_pre_body(x_ref, win_ref, bin_ref, wab_ref, bab_ref,
                 h_ref, a_ref, hb_ref):
    h = jnp.maximum(x_ref[...] @ win_ref[...] + bin_ref[...], 0.0)
    h_ref[...] = h
    a_ref[...] = h @ wab_ref[...] + bab_ref[...]
    hb_ref[...] = h.astype(jnp.bfloat16)


_tc_pre = pl.pallas_call(
    _tc_pre_body,
    out_shape=[jax.ShapeDtypeStruct((N_PAD, D), jnp.float32),
               jax.ShapeDtypeStruct((N_PAD, 2), jnp.float32),
               jax.ShapeDtypeStruct((N_PAD, D), jnp.bfloat16)],
)


def _tc_layer_body(u_ref, g_ref, h_ref, wm_ref, bm_ref, wr_ref, br_ref,
                   gam_ref, bet_ref, wab_ref, bab_ref, hn_ref, a_ref, hb_ref):
    u = u_ref[0] + u_ref[1]
    gcol = g_ref[0] + g_ref[1]
    h = h_ref[...]
    z = (u @ wm_ref[...] + gcol * bm_ref[...]
         + h @ wr_ref[...] + br_ref[...])
    rows = lax.broadcasted_iota(jnp.int32, (N_PAD, 1), 0)
    mask = rows < N
    zm = jnp.where(mask, z, 0.0)
    mean = jnp.sum(zm, axis=0, keepdims=True) * (1.0 / N)
    dev = jnp.where(mask, z - mean, 0.0)
    var = jnp.sum(dev * dev, axis=0, keepdims=True) * (1.0 / N)
    zn = (z - mean) * lax.rsqrt(var + 1e-5) * gam_ref[...] + bet_ref[...]
    hn = jnp.maximum(zn, 0.0)
    hn_ref[...] = hn
    a_ref[...] = hn @ wab_ref[...] + bab_ref[...]
    hb_ref[...] = hn.astype(jnp.bfloat16)


_tc_layer = pl.pallas_call(
    _tc_layer_body,
    out_shape=[jax.ShapeDtypeStruct((N_PAD, D), jnp.float32),
               jax.ShapeDtypeStruct((N_PAD, 2), jnp.float32),
               jax.ShapeDtypeStruct((N_PAD, D), jnp.bfloat16)],
)


def _tc_final_body(h_ref, batch_ref, w1_ref, b1_ref, w2_ref, b2_ref, out_ref):
    gid = lax.broadcasted_iota(jnp.int32, (NUM_GRAPHS, N_PAD), 0)
    onehot_t = (gid == batch_ref[...]).astype(jnp.float32)
    pooled = onehot_t @ h_ref[...]
    hid = jnp.maximum(pooled @ w1_ref[...] + b1_ref[...], 0.0)
    out_ref[...] = (hid @ w2_ref[...] + b2_ref[...]) * 0.5


_tc_final = pl.pallas_call(
    _tc_final_body,
    out_shape=jax.ShapeDtypeStruct((NUM_GRAPHS, NUM_CLASSES), jnp.float32),
)


def kernel(x, edge_index, batch, W_in, b_in, Wa, ba, Wm, bm, Wr, br,
           gamma, beta, W1, b1, W2, b2):
    f32 = jnp.float32
    src = edge_index[0].astype(jnp.int32)
    dst = edge_index[1].astype(jnp.int32)
    pad_e = E_PAD - E
    packed = src | (dst << 14)
    packed_p = jnp.concatenate(
        [packed, jnp.full((pad_e,), N << 14, jnp.int32)]).reshape(NW, NCH // 2, 2 * CH)
    x_p = jnp.concatenate([x.astype(f32), jnp.zeros((N_PAD - N, D), f32)], axis=0)
    batch2 = jnp.concatenate(
        [batch.astype(jnp.int32),
         jnp.full((N_PAD - N,), NUM_GRAPHS, jnp.int32)]).reshape(1, N_PAD)

    wabs, babs = [], []
    for i in range(L):
        wabs.append(jnp.concatenate([Wa[i, :D, :], Wa[i, D:, :]], axis=1))
        babs.append(jnp.concatenate(
            [jnp.zeros((1,), f32), ba[i]]).reshape(1, 2))
    wabs.append(jnp.zeros((D, 2), f32))
    babs.append(jnp.zeros((1, 2), f32))

    sigma = np.concatenate([
        np.concatenate([np.arange(32 * q, 32 * q + 32, 2),
                        np.arange(32 * q + 1, 32 * q + 32, 2)])
        for q in range(D // 32)])
    h, A, hb = _tc_pre(x_p, W_in, b_in.reshape(1, D), wabs[0], babs[0])
    for i in range(L):
        a_s = A[:, 0]
        a_d = A[:, 1]
        hbi = lax.bitcast_convert_type(
            hb.reshape(N_PAD, D // 2, 2), jnp.int32)
        u_part, g_part = _make_sc_edges()(hbi, a_s, a_d, packed_p)
        h, A, hb = _tc_layer(u_part, g_part.reshape(NC, N_PAD, 1), h,
                         Wm[i][sigma, :], bm[i].reshape(1, D),
                         Wr[i], br[i].reshape(1, D),
                         gamma[i].reshape(1, D), beta[i].reshape(1, D),
                         wabs[i + 1], babs[i + 1])
    logits = _tc_final(h, batch2, W1, b1.reshape(1, D // 2),
                       W2, b2.reshape(1, NUM_CLASSES))
    return logits
